# baseline (speedup 1.0000x reference)
import functools

import jax
import jax.numpy as jnp
from jax import lax
from jax.experimental import pallas as pl
from jax.experimental.pallas import tpu as pltpu
from jax.experimental.pallas import tpu_sc as plsc

VOCAB = 100000
HIDDEN = 768
MAX_POS = 512
EPS = 1e-12

TOKEN_BLOCK = 1024
NUM_WORKERS = 32
CHUNK = 80
SLICES = 4
EXT = MAX_POS + 8


def _sc_gather(word_emb, flat_ids, n_tokens):
    per_w = n_tokens // NUM_WORKERS
    n_chunks = per_w // CHUNK
    assert per_w % CHUNK == 0 and n_chunks % 2 == 0
    mesh = plsc.VectorSubcoreMesh(core_axis_name="c", subcore_axis_name="s")

    @functools.partial(
        pl.kernel,
        out_type=jax.ShapeDtypeStruct((n_tokens, HIDDEN), jnp.float32),
        mesh=mesh,
        scratch_types=[
            pltpu.VMEM((per_w,), jnp.int32),
            pltpu.VMEM((CHUNK, HIDDEN), jnp.float32),
            pltpu.VMEM((CHUNK, HIDDEN), jnp.float32),
            pltpu.SemaphoreType.DMA,
            pltpu.SemaphoreType.DMA,
            pltpu.SemaphoreType.DMA,
            pltpu.SemaphoreType.DMA,
        ],
    )
    def gather_kernel(table_hbm, idx_hbm, out_hbm, idx_v, buf_a, buf_b,
                      gs_a, gs_b, ws_a, ws_b):
        wid = lax.axis_index("s") * 2 + lax.axis_index("c")
        base = wid * per_w
        pltpu.sync_copy(idx_hbm.at[pl.ds(base, per_w)], idx_v)

        def gather(c, buf, sem):
            return pltpu.make_async_copy(
                table_hbm.at[idx_v.at[pl.ds(c * CHUNK, CHUNK)]], buf, sem)

        def wback(c, buf, sem):
            return pltpu.make_async_copy(
                buf, out_hbm.at[pl.ds(base + c * CHUNK, CHUNK)], sem)

        gather(0, buf_a, gs_a).start()
        gather(1, buf_b, gs_b).start()

        @pl.loop(0, n_chunks, step=2)
        def _(c):
            gather(c, buf_a, gs_a).wait()
            wback(c, buf_a, ws_a).start()
            gather(c + 1, buf_b, gs_b).wait()
            wback(c + 1, buf_b, ws_b).start()

            @pl.when(c + 2 < n_chunks)
            def _():
                wback(c, buf_a, ws_a).wait()
                gather(c + 2, buf_a, gs_a).start()
                wback(c + 1, buf_b, ws_b).wait()
                gather(c + 3, buf_b, gs_b).start()

        wback(n_chunks - 2, buf_a, ws_a).wait()
        wback(n_chunks - 1, buf_b, ws_b).wait()

    return gather_kernel(word_emb, flat_ids)


def _tc_body(dst_ref, w_ref, pid_ref, tid_ref, ext_ref, gamma_ref, beta_ref,
             out_ref):
    del dst_ref
    pid = pid_ref[0]
    tid = tid_ref[0]
    iota = lax.broadcasted_iota(jnp.int32, (EXT, TOKEN_BLOCK), 0)
    oh = ((iota == pid) | (iota - MAX_POS == tid)).astype(jnp.bfloat16)
    pt = lax.dot_general(oh, ext_ref[...], (((0,), (0,)), ((), ())),
                         preferred_element_type=jnp.float32)
    x = w_ref[...] + pt
    s1 = jnp.sum(x, axis=1, keepdims=True)
    s2 = jnp.sum(x * x, axis=1, keepdims=True)
    m = s1 * (1.0 / HIDDEN)
    var = s2 * (1.0 / HIDDEN) - m * m
    r = lax.rsqrt(var + EPS)
    out_ref[...] = (x - m) * r * gamma_ref[...] + beta_ref[...]


def _tc_fuse_slice(dst, w_rows, pid_s, tid_s, ext_table, gamma2, beta2,
                   block_off, n_tokens, slice_tokens):
    n_blocks = slice_tokens // TOKEN_BLOCK
    pid3 = pid_s.reshape(n_blocks, 1, TOKEN_BLOCK)
    tid3 = tid_s.reshape(n_blocks, 1, TOKEN_BLOCK)

    return pl.pallas_call(
        _tc_body,
        grid=(n_blocks,),
        in_specs=[
            pl.BlockSpec(memory_space=pl.ANY),
            pl.BlockSpec((TOKEN_BLOCK, HIDDEN), lambda i: (i, 0)),
            pl.BlockSpec((1, 1, TOKEN_BLOCK), lambda i: (i, 0, 0)),
            pl.BlockSpec((1, 1, TOKEN_BLOCK), lambda i: (i, 0, 0)),
            pl.BlockSpec((EXT, HIDDEN), lambda i: (0, 0)),
            pl.BlockSpec((1, HIDDEN), lambda i: (0, 0)),
            pl.BlockSpec((1, HIDDEN), lambda i: (0, 0)),
        ],
        out_specs=pl.BlockSpec((TOKEN_BLOCK, HIDDEN),
                               lambda i: (i + block_off, 0)),
        out_shape=jax.ShapeDtypeStruct((n_tokens, HIDDEN), jnp.float32),
        input_output_aliases={0: 0},
    )(dst, w_rows, pid3, tid3, ext_table, gamma2, beta2)


@jax.jit
def kernel(input_ids, position_ids, token_type_ids, word_emb, pos_emb,
           type_emb, gamma, beta):
    b, l = input_ids.shape
    n_tokens = b * l
    slice_tokens = n_tokens // SLICES
    flat_ids = input_ids.reshape(-1)
    pid = position_ids.reshape(-1)
    tid = token_type_ids.reshape(-1)
    ext_table = jnp.concatenate(
        [pos_emb, type_emb, jnp.zeros((EXT - MAX_POS - 2, HIDDEN),
                                      jnp.float32)],
        axis=0).astype(jnp.bfloat16)
    gamma2 = gamma.reshape(1, HIDDEN)
    beta2 = beta.reshape(1, HIDDEN)

    blocks_per_slice = slice_tokens // TOKEN_BLOCK
    out = jnp.empty((n_tokens, HIDDEN), jnp.float32)
    for s in range(SLICES):
        lo = s * slice_tokens
        hi = lo + slice_tokens
        w_s = _sc_gather(word_emb, flat_ids[lo:hi], slice_tokens)
        out = _tc_fuse_slice(out, w_s, pid[lo:hi], tid[lo:hi], ext_table,
                             gamma2, beta2, s * blocks_per_slice, n_tokens,
                             slice_tokens)
    return out.reshape(b, l, HIDDEN)


# speedup vs baseline: 1.3416x; 1.3416x over previous
import functools

import jax
import jax.numpy as jnp
from jax import lax
from jax.experimental import pallas as pl
from jax.experimental.pallas import tpu as pltpu
from jax.experimental.pallas import tpu_sc as plsc

VOCAB = 100000
HIDDEN = 768
MAX_POS = 512
EPS = 1e-12

TOKEN_BLOCK = 1024
NUM_WORKERS = 32
CHUNK = 80
SLICES = 4
EXT = MAX_POS + 8


def _sc_gather(word_emb, flat_ids, n_tokens):
    per_w = n_tokens // NUM_WORKERS
    n_chunks = per_w // CHUNK
    assert per_w % CHUNK == 0 and n_chunks % 2 == 0
    mesh = plsc.VectorSubcoreMesh(core_axis_name="c", subcore_axis_name="s")

    @functools.partial(
        pl.kernel,
        out_type=jax.ShapeDtypeStruct((n_tokens, HIDDEN), jnp.float32),
        mesh=mesh,
        scratch_types=[
            pltpu.VMEM((per_w,), jnp.int32),
            pltpu.VMEM((CHUNK, HIDDEN), jnp.float32),
            pltpu.VMEM((CHUNK, HIDDEN), jnp.float32),
            pltpu.SemaphoreType.DMA,
            pltpu.SemaphoreType.DMA,
            pltpu.SemaphoreType.DMA,
            pltpu.SemaphoreType.DMA,
        ],
    )
    def gather_kernel(table_hbm, idx_hbm, out_hbm, idx_v, buf_a, buf_b,
                      gs_a, gs_b, ws_a, ws_b):
        wid = lax.axis_index("s") * 2 + lax.axis_index("c")
        base = wid * per_w
        pltpu.sync_copy(idx_hbm.at[pl.ds(base, per_w)], idx_v)

        def gather(c, buf, sem):
            return pltpu.make_async_copy(
                table_hbm.at[idx_v.at[pl.ds(c * CHUNK, CHUNK)]], buf, sem)

        def wback(c, buf, sem):
            return pltpu.make_async_copy(
                buf, out_hbm.at[pl.ds(base + c * CHUNK, CHUNK)], sem)

        gather(0, buf_a, gs_a).start()
        gather(1, buf_b, gs_b).start()

        @pl.loop(0, n_chunks, step=2)
        def _(c):
            gather(c, buf_a, gs_a).wait()
            wback(c, buf_a, ws_a).start()
            gather(c + 1, buf_b, gs_b).wait()
            wback(c + 1, buf_b, ws_b).start()

            @pl.when(c + 2 < n_chunks)
            def _():
                wback(c, buf_a, ws_a).wait()
                gather(c + 2, buf_a, gs_a).start()
                wback(c + 1, buf_b, ws_b).wait()
                gather(c + 3, buf_b, gs_b).start()

        wback(n_chunks - 2, buf_a, ws_a).wait()
        wback(n_chunks - 1, buf_b, ws_b).wait()

    return gather_kernel(word_emb, flat_ids)


def _tc_body(*refs):
    if len(refs) == 8:
        refs = refs[1:]
    w_ref, pid_ref, tid_ref, ext_ref, gamma_ref, beta_ref, out_ref = refs
    pid = pid_ref[0]
    tid = tid_ref[0]
    iota = lax.broadcasted_iota(jnp.int32, (EXT, TOKEN_BLOCK), 0)
    oh = ((iota == pid) | (iota - MAX_POS == tid)).astype(jnp.bfloat16)
    pt = lax.dot_general(oh, ext_ref[...], (((0,), (0,)), ((), ())),
                         preferred_element_type=jnp.float32)
    x = w_ref[...] + pt
    s1 = jnp.sum(x, axis=1, keepdims=True)
    s2 = jnp.sum(x * x, axis=1, keepdims=True)
    m = s1 * (1.0 / HIDDEN)
    var = s2 * (1.0 / HIDDEN) - m * m
    r = lax.rsqrt(var + EPS)
    out_ref[...] = (x - m) * r * gamma_ref[...] + beta_ref[...]


def _tc_fuse_slice(dst, w_rows, pid_s, tid_s, ext_table, gamma2, beta2,
                   block_off, n_tokens, slice_tokens):
    n_blocks = slice_tokens // TOKEN_BLOCK
    pid3 = pid_s.reshape(n_blocks, 1, TOKEN_BLOCK)
    tid3 = tid_s.reshape(n_blocks, 1, TOKEN_BLOCK)

    in_specs = [
        pl.BlockSpec((TOKEN_BLOCK, HIDDEN), lambda i: (i, 0)),
        pl.BlockSpec((1, 1, TOKEN_BLOCK), lambda i: (i, 0, 0)),
        pl.BlockSpec((1, 1, TOKEN_BLOCK), lambda i: (i, 0, 0)),
        pl.BlockSpec((EXT, HIDDEN), lambda i: (0, 0)),
        pl.BlockSpec((1, HIDDEN), lambda i: (0, 0)),
        pl.BlockSpec((1, HIDDEN), lambda i: (0, 0)),
    ]
    args = (w_rows, pid3, tid3, ext_table, gamma2, beta2)
    aliases = {}
    if dst is not None:
        in_specs = [pl.BlockSpec(memory_space=pl.ANY)] + in_specs
        args = (dst,) + args
        aliases = {0: 0}

    return pl.pallas_call(
        _tc_body,
        grid=(n_blocks,),
        in_specs=in_specs,
        out_specs=pl.BlockSpec((TOKEN_BLOCK, HIDDEN),
                               lambda i: (i + block_off, 0)),
        out_shape=jax.ShapeDtypeStruct((n_tokens, HIDDEN), jnp.float32),
        input_output_aliases=aliases,
    )(*args)


@jax.jit
def kernel(input_ids, position_ids, token_type_ids, word_emb, pos_emb,
           type_emb, gamma, beta):
    b, l = input_ids.shape
    n_tokens = b * l
    slice_tokens = n_tokens // SLICES
    flat_ids = input_ids.reshape(-1)
    pid = position_ids.reshape(-1)
    tid = token_type_ids.reshape(-1)
    ext_table = jnp.concatenate(
        [pos_emb, type_emb, jnp.zeros((EXT - MAX_POS - 2, HIDDEN),
                                      jnp.float32)],
        axis=0).astype(jnp.bfloat16)
    gamma2 = gamma.reshape(1, HIDDEN)
    beta2 = beta.reshape(1, HIDDEN)

    blocks_per_slice = slice_tokens // TOKEN_BLOCK
    out = None
    for s in range(SLICES):
        lo = s * slice_tokens
        hi = lo + slice_tokens
        w_s = _sc_gather(word_emb, flat_ids[lo:hi], slice_tokens)
        out = _tc_fuse_slice(out, w_s, pid[lo:hi], tid[lo:hi], ext_table,
                             gamma2, beta2, s * blocks_per_slice, n_tokens,
                             slice_tokens)
    return out.reshape(b, l, HIDDEN)


# speedup vs baseline: 1.3530x; 1.0085x over previous
import functools

import jax
import jax.numpy as jnp
from jax import lax
from jax.experimental import pallas as pl
from jax.experimental.pallas import tpu as pltpu
from jax.experimental.pallas import tpu_sc as plsc

VOCAB = 100000
HIDDEN = 768
MAX_POS = 512
EPS = 1e-12

TOKEN_BLOCK = 1024
NUM_WORKERS = 32
CHUNK = 80
SLICES = 8
EXT = MAX_POS + 8


def _sc_gather(word_emb, flat_ids, n_tokens):
    per_w = n_tokens // NUM_WORKERS
    n_chunks = per_w // CHUNK
    assert per_w % CHUNK == 0 and n_chunks % 2 == 0
    mesh = plsc.VectorSubcoreMesh(core_axis_name="c", subcore_axis_name="s")

    @functools.partial(
        pl.kernel,
        out_type=jax.ShapeDtypeStruct((n_tokens, HIDDEN), jnp.float32),
        mesh=mesh,
        scratch_types=[
            pltpu.VMEM((per_w,), jnp.int32),
            pltpu.VMEM((CHUNK, HIDDEN), jnp.float32),
            pltpu.VMEM((CHUNK, HIDDEN), jnp.float32),
            pltpu.SemaphoreType.DMA,
            pltpu.SemaphoreType.DMA,
            pltpu.SemaphoreType.DMA,
            pltpu.SemaphoreType.DMA,
        ],
    )
    def gather_kernel(table_hbm, idx_hbm, out_hbm, idx_v, buf_a, buf_b,
                      gs_a, gs_b, ws_a, ws_b):
        wid = lax.axis_index("s") * 2 + lax.axis_index("c")
        base = wid * per_w
        pltpu.sync_copy(idx_hbm.at[pl.ds(base, per_w)], idx_v)

        def gather(c, buf, sem):
            return pltpu.make_async_copy(
                table_hbm.at[idx_v.at[pl.ds(c * CHUNK, CHUNK)]], buf, sem)

        def wback(c, buf, sem):
            return pltpu.make_async_copy(
                buf, out_hbm.at[pl.ds(base + c * CHUNK, CHUNK)], sem)

        gather(0, buf_a, gs_a).start()
        gather(1, buf_b, gs_b).start()

        @pl.loop(0, n_chunks, step=2)
        def _(c):
            gather(c, buf_a, gs_a).wait()
            wback(c, buf_a, ws_a).start()
            gather(c + 1, buf_b, gs_b).wait()
            wback(c + 1, buf_b, ws_b).start()

            @pl.when(c + 2 < n_chunks)
            def _():
                wback(c, buf_a, ws_a).wait()
                gather(c + 2, buf_a, gs_a).start()
                wback(c + 1, buf_b, ws_b).wait()
                gather(c + 3, buf_b, gs_b).start()

        wback(n_chunks - 2, buf_a, ws_a).wait()
        wback(n_chunks - 1, buf_b, ws_b).wait()

    return gather_kernel(word_emb, flat_ids)


def _tc_body(*refs):
    if len(refs) == 8:
        refs = refs[1:]
    w_ref, pid_ref, tid_ref, ext_ref, gamma_ref, beta_ref, out_ref = refs
    pid = pid_ref[0]
    tid = tid_ref[0]
    iota = lax.broadcasted_iota(jnp.int32, (EXT, TOKEN_BLOCK), 0)
    oh = ((iota == pid) | (iota - MAX_POS == tid)).astype(jnp.bfloat16)
    pt = lax.dot_general(oh, ext_ref[...], (((0,), (0,)), ((), ())),
                         preferred_element_type=jnp.float32)
    x = w_ref[...] + pt
    s1 = jnp.sum(x, axis=1, keepdims=True)
    s2 = jnp.sum(x * x, axis=1, keepdims=True)
    m = s1 * (1.0 / HIDDEN)
    var = s2 * (1.0 / HIDDEN) - m * m
    r = lax.rsqrt(var + EPS)
    out_ref[...] = (x - m) * r * gamma_ref[...] + beta_ref[...]


def _tc_fuse_slice(dst, w_rows, pid_s, tid_s, ext_table, gamma2, beta2,
                   block_off, n_tokens, slice_tokens):
    n_blocks = slice_tokens // TOKEN_BLOCK
    pid3 = pid_s.reshape(n_blocks, 1, TOKEN_BLOCK)
    tid3 = tid_s.reshape(n_blocks, 1, TOKEN_BLOCK)

    in_specs = [
        pl.BlockSpec((TOKEN_BLOCK, HIDDEN), lambda i: (i, 0)),
        pl.BlockSpec((1, 1, TOKEN_BLOCK), lambda i: (i, 0, 0)),
        pl.BlockSpec((1, 1, TOKEN_BLOCK), lambda i: (i, 0, 0)),
        pl.BlockSpec((EXT, HIDDEN), lambda i: (0, 0)),
        pl.BlockSpec((1, HIDDEN), lambda i: (0, 0)),
        pl.BlockSpec((1, HIDDEN), lambda i: (0, 0)),
    ]
    args = (w_rows, pid3, tid3, ext_table, gamma2, beta2)
    aliases = {}
    if dst is not None:
        in_specs = [pl.BlockSpec(memory_space=pl.ANY)] + in_specs
        args = (dst,) + args
        aliases = {0: 0}

    return pl.pallas_call(
        _tc_body,
        grid=(n_blocks,),
        in_specs=in_specs,
        out_specs=pl.BlockSpec((TOKEN_BLOCK, HIDDEN),
                               lambda i: (i + block_off, 0)),
        out_shape=jax.ShapeDtypeStruct((n_tokens, HIDDEN), jnp.float32),
        input_output_aliases=aliases,
    )(*args)


@jax.jit
def kernel(input_ids, position_ids, token_type_ids, word_emb, pos_emb,
           type_emb, gamma, beta):
    b, l = input_ids.shape
    n_tokens = b * l
    slice_tokens = n_tokens // SLICES
    flat_ids = input_ids.reshape(-1)
    pid = position_ids.reshape(-1)
    tid = token_type_ids.reshape(-1)
    ext_table = jnp.concatenate(
        [pos_emb, type_emb, jnp.zeros((EXT - MAX_POS - 2, HIDDEN),
                                      jnp.float32)],
        axis=0).astype(jnp.bfloat16)
    gamma2 = gamma.reshape(1, HIDDEN)
    beta2 = beta.reshape(1, HIDDEN)

    blocks_per_slice = slice_tokens // TOKEN_BLOCK
    out = None
    for s in range(SLICES):
        lo = s * slice_tokens
        hi = lo + slice_tokens
        w_s = _sc_gather(word_emb, flat_ids[lo:hi], slice_tokens)
        out = _tc_fuse_slice(out, w_s, pid[lo:hi], tid[lo:hi], ext_table,
                             gamma2, beta2, s * blocks_per_slice, n_tokens,
                             slice_tokens)
    return out.reshape(b, l, HIDDEN)


# speedup vs baseline: 1.5133x; 1.1185x over previous
import functools

import jax
import jax.numpy as jnp
from jax import lax
from jax.experimental import pallas as pl
from jax.experimental.pallas import tpu as pltpu
from jax.experimental.pallas import tpu_sc as plsc

VOCAB = 100000
HIDDEN = 768
MAX_POS = 512
EPS = 1e-12

TOKEN_BLOCK = 1024
NUM_WORKERS = 32
CHUNK = 80
SLICES = 8
EXT = MAX_POS + 8


HALF = HIDDEN // 2


def _convert_body(in_ref, out_ref):
    u = lax.bitcast_convert_type(in_ref[...], jnp.uint32)
    r = (u + jnp.uint32(0x7FFF) + ((u >> 16) & jnp.uint32(1))) >> 16
    lo = r[:, :HALF]
    hi = r[:, HALF:]
    out_ref[...] = lax.bitcast_convert_type((hi << 16) | lo, jnp.int32)


def _to_bf16(word_emb):
    rows = word_emb.shape[0]
    block = 4000
    return pl.pallas_call(
        _convert_body,
        grid=(rows // block,),
        in_specs=[pl.BlockSpec((block, HIDDEN), lambda i: (i, 0))],
        out_specs=pl.BlockSpec((block, HALF), lambda i: (i, 0)),
        out_shape=jax.ShapeDtypeStruct((rows, HALF), jnp.int32),
    )(word_emb)


def _sc_gather(table, flat_ids, n_tokens):
    width = table.shape[1]
    per_w = n_tokens // NUM_WORKERS
    n_chunks = per_w // CHUNK
    assert per_w % CHUNK == 0 and n_chunks % 2 == 0
    mesh = plsc.VectorSubcoreMesh(core_axis_name="c", subcore_axis_name="s")

    @functools.partial(
        pl.kernel,
        out_type=jax.ShapeDtypeStruct((n_tokens, width), table.dtype),
        mesh=mesh,
        scratch_types=[
            pltpu.VMEM((per_w,), jnp.int32),
            pltpu.VMEM((CHUNK, width), table.dtype),
            pltpu.VMEM((CHUNK, width), table.dtype),
            pltpu.SemaphoreType.DMA,
            pltpu.SemaphoreType.DMA,
            pltpu.SemaphoreType.DMA,
            pltpu.SemaphoreType.DMA,
        ],
    )
    def gather_kernel(table_hbm, idx_hbm, out_hbm, idx_v, buf_a, buf_b,
                      gs_a, gs_b, ws_a, ws_b):
        wid = lax.axis_index("s") * 2 + lax.axis_index("c")
        base = wid * per_w
        pltpu.sync_copy(idx_hbm.at[pl.ds(base, per_w)], idx_v)

        def gather(c, buf, sem):
            return pltpu.make_async_copy(
                table_hbm.at[idx_v.at[pl.ds(c * CHUNK, CHUNK)]], buf, sem)

        def wback(c, buf, sem):
            return pltpu.make_async_copy(
                buf, out_hbm.at[pl.ds(base + c * CHUNK, CHUNK)], sem)

        gather(0, buf_a, gs_a).start()
        gather(1, buf_b, gs_b).start()

        @pl.loop(0, n_chunks, step=2)
        def _(c):
            gather(c, buf_a, gs_a).wait()
            wback(c, buf_a, ws_a).start()
            gather(c + 1, buf_b, gs_b).wait()
            wback(c + 1, buf_b, ws_b).start()

            @pl.when(c + 2 < n_chunks)
            def _():
                wback(c, buf_a, ws_a).wait()
                gather(c + 2, buf_a, gs_a).start()
                wback(c + 1, buf_b, ws_b).wait()
                gather(c + 3, buf_b, gs_b).start()

        wback(n_chunks - 2, buf_a, ws_a).wait()
        wback(n_chunks - 1, buf_b, ws_b).wait()

    return gather_kernel(table, flat_ids)


def _tc_body(*refs):
    if len(refs) == 8:
        refs = refs[1:]
    w_ref, pid_ref, tid_ref, ext_ref, gamma_ref, beta_ref, out_ref = refs
    pid = pid_ref[0]
    tid = tid_ref[0]
    iota = lax.broadcasted_iota(jnp.int32, (EXT, TOKEN_BLOCK), 0)
    oh = ((iota == pid) | (iota - MAX_POS == tid)).astype(jnp.bfloat16)
    pt = lax.dot_general(oh, ext_ref[...], (((0,), (0,)), ((), ())),
                         preferred_element_type=jnp.float32)
    wu = lax.bitcast_convert_type(w_ref[...], jnp.uint32)
    w_lo = lax.bitcast_convert_type(wu << 16, jnp.float32)
    w_hi = lax.bitcast_convert_type(wu & jnp.uint32(0xFFFF0000), jnp.float32)
    x = jnp.concatenate([w_lo, w_hi], axis=1) + pt
    s1 = jnp.sum(x, axis=1, keepdims=True)
    s2 = jnp.sum(x * x, axis=1, keepdims=True)
    m = s1 * (1.0 / HIDDEN)
    var = s2 * (1.0 / HIDDEN) - m * m
    r = lax.rsqrt(var + EPS)
    out_ref[...] = (x - m) * r * gamma_ref[...] + beta_ref[...]


def _tc_fuse_slice(dst, w_rows, pid_s, tid_s, ext_table, gamma2, beta2,
                   block_off, n_tokens, slice_tokens):
    n_blocks = slice_tokens // TOKEN_BLOCK
    pid3 = pid_s.reshape(n_blocks, 1, TOKEN_BLOCK)
    tid3 = tid_s.reshape(n_blocks, 1, TOKEN_BLOCK)

    in_specs = [
        pl.BlockSpec((TOKEN_BLOCK, HALF), lambda i: (i, 0)),
        pl.BlockSpec((1, 1, TOKEN_BLOCK), lambda i: (i, 0, 0)),
        pl.BlockSpec((1, 1, TOKEN_BLOCK), lambda i: (i, 0, 0)),
        pl.BlockSpec((EXT, HIDDEN), lambda i: (0, 0)),
        pl.BlockSpec((1, HIDDEN), lambda i: (0, 0)),
        pl.BlockSpec((1, HIDDEN), lambda i: (0, 0)),
    ]
    args = (w_rows, pid3, tid3, ext_table, gamma2, beta2)
    aliases = {}
    if dst is not None:
        in_specs = [pl.BlockSpec(memory_space=pl.ANY)] + in_specs
        args = (dst,) + args
        aliases = {0: 0}

    return pl.pallas_call(
        _tc_body,
        grid=(n_blocks,),
        in_specs=in_specs,
        out_specs=pl.BlockSpec((TOKEN_BLOCK, HIDDEN),
                               lambda i: (i + block_off, 0)),
        out_shape=jax.ShapeDtypeStruct((n_tokens, HIDDEN), jnp.float32),
        input_output_aliases=aliases,
    )(*args)


@jax.jit
def kernel(input_ids, position_ids, token_type_ids, word_emb, pos_emb,
           type_emb, gamma, beta):
    b, l = input_ids.shape
    n_tokens = b * l
    slice_tokens = n_tokens // SLICES
    flat_ids = input_ids.reshape(-1)
    pid = position_ids.reshape(-1)
    tid = token_type_ids.reshape(-1)
    ext_table = jnp.concatenate(
        [pos_emb, type_emb, jnp.zeros((EXT - MAX_POS - 2, HIDDEN),
                                      jnp.float32)],
        axis=0).astype(jnp.bfloat16)
    gamma2 = gamma.reshape(1, HIDDEN)
    beta2 = beta.reshape(1, HIDDEN)

    blocks_per_slice = slice_tokens // TOKEN_BLOCK
    table_bf = _to_bf16(word_emb)
    out = None
    for s in range(SLICES):
        lo = s * slice_tokens
        hi = lo + slice_tokens
        w_s = _sc_gather(table_bf, flat_ids[lo:hi], slice_tokens)
        out = _tc_fuse_slice(out, w_s, pid[lo:hi], tid[lo:hi], ext_table,
                             gamma2, beta2, s * blocks_per_slice, n_tokens,
                             slice_tokens)
    return out.reshape(b, l, HIDDEN)
